# baseline (speedup 1.0000x reference)
import functools

import jax
import jax.numpy as jnp
from jax import lax
from jax.experimental import pallas as pl
from jax.experimental.pallas import tpu as pltpu
from jax.experimental.pallas import tpu_sc as plsc

L = 16
NW = 32
SLOTS = 43
PER_TILE = SLOTS * L
NPAD = NW * PER_TILE
GP = 208
NG = 200


def _sc_body(ax0_h, ay0_h, ax1_h, ay1_h, gx0_h, gy0_h, gx1_h, gy1_h, out_h,
             ax0_v, ay0_v, ax1_v, ay1_v, g0_v, g1_v, g2_v, g3_v,
             gcx_v, gcy_v, glv_v, out_v):
    nc = 2
    wid = lax.axis_index("s") * nc + lax.axis_index("c")

    pltpu.sync_copy(ax0_h.at[wid], ax0_v)
    pltpu.sync_copy(ay0_h.at[wid], ay0_v)
    pltpu.sync_copy(ax1_h.at[wid], ax1_v)
    pltpu.sync_copy(ay1_h.at[wid], ay1_v)
    pltpu.sync_copy(gx0_h, g0_v)
    pltpu.sync_copy(gy0_h, g1_v)
    pltpu.sync_copy(gx1_h, g2_v)
    pltpu.sync_copy(gy1_h, g3_v)

    for k in range(GP // L):
        s = pl.ds(k * L, L)
        x0 = g0_v[s]
        y0 = g1_v[s]
        x1 = g2_v[s]
        y1 = g3_v[s]
        area = (x1 - x0) * (y1 - y0)
        lv = jnp.zeros((L,), jnp.float32)
        for thr in (1024.0, 4096.0, 16384.0, 65536.0):
            lv = lv + jnp.where(area >= thr, 1.0, 0.0).astype(jnp.float32)
        lv = jnp.where(area >= 262144.0, 0.0, lv)
        gcx_v[s] = (x0 + x1) * 0.5
        gcy_v[s] = (y0 + y1) * 0.5
        glv_v[s] = lv

    lvl42 = jnp.where(wid < 16, 3.0, jnp.where(wid < 20, 4.0, 99.0)).astype(jnp.float32)

    for i in range(SLOTS):
        s = pl.ds(i * L, L)
        a0 = ax0_v[s]
        a1 = ay0_v[s]
        a2 = ax1_v[s]
        a3 = ay1_v[s]
        if i < 32:
            alvl = 0.0
        elif i < 40:
            alvl = 1.0
        elif i < 42:
            alvl = 2.0
        else:
            alvl = lvl42

        def body(g, assign, a0=a0, a1=a1, a2=a2, a3=a3, alvl=alvl):
            idx = jnp.full((L,), g, dtype=jnp.int32)
            bcx = plsc.load_gather(gcx_v, [idx])
            bcy = plsc.load_gather(gcy_v, [idx])
            blv = plsc.load_gather(glv_v, [idx])
            m = ((bcx > a0) & (bcy > a1) & (bcx < a2) & (bcy < a3)
                 & (blv == alvl))
            return jnp.where(m, idx, assign)

        assign = lax.fori_loop(0, NG, body, jnp.full((L,), -2, jnp.int32))
        out_v[s] = assign

    pltpu.sync_copy(out_v, out_h.at[wid])


@jax.jit
def kernel(anchor, gts):
    n = anchor.shape[0]
    a = jnp.zeros((NPAD, 4), jnp.float32).at[:n].set(anchor)
    a = a.reshape(SLOTS, NW, L, 4).transpose(1, 0, 2, 3).reshape(NW, PER_TILE, 4)
    ax0, ay0, ax1, ay1 = (a[:, :, j] for j in range(4))
    g = jnp.full((GP, 4), 2e9, jnp.float32).at[: gts.shape[0]].set(gts)
    gx0, gy0, gx1, gy1 = (g[:, j] for j in range(4))

    mesh = plsc.VectorSubcoreMesh(core_axis_name="c", subcore_axis_name="s")
    run = pl.kernel(
        _sc_body,
        mesh=mesh,
        compiler_params=pltpu.CompilerParams(needs_layout_passes=False),
        out_type=jax.ShapeDtypeStruct((NW, PER_TILE), jnp.int32),
        scratch_types=[
            pltpu.VMEM((PER_TILE,), jnp.float32),
            pltpu.VMEM((PER_TILE,), jnp.float32),
            pltpu.VMEM((PER_TILE,), jnp.float32),
            pltpu.VMEM((PER_TILE,), jnp.float32),
            pltpu.VMEM((GP,), jnp.float32),
            pltpu.VMEM((GP,), jnp.float32),
            pltpu.VMEM((GP,), jnp.float32),
            pltpu.VMEM((GP,), jnp.float32),
            pltpu.VMEM((GP,), jnp.float32),
            pltpu.VMEM((GP,), jnp.float32),
            pltpu.VMEM((GP,), jnp.float32),
            pltpu.VMEM((PER_TILE,), jnp.int32),
        ],
    )
    out = run(ax0, ay0, ax1, ay1, gx0, gy0, gx1, gy1)
    out = out.reshape(NW, SLOTS, L).transpose(1, 0, 2).reshape(NPAD)[:n]
    return out.astype(jnp.int64)


# speedup vs baseline: 1.8552x; 1.8552x over previous
import jax
import jax.numpy as jnp
from jax import lax
from jax.experimental import pallas as pl
from jax.experimental.pallas import tpu as pltpu
from jax.experimental.pallas import tpu_sc as plsc

L = 16
NW = 32
N = 21824
NG = 200
GP = 208
GF = 4 * GP
B = 224
PER_TILE = 688
SENT = 2.0e9

CHUNKS = ((0, 512, 512, 0), (16384, 128, 128, 512),
          (20480, 32, 32, 640), (21504, 16, 16, 672))


def _sc_body(anchor_h, gts_h, out_h,
             av, gv, bcx, bcy, bgi, outv,
             sem_g, sem_a0, sem_a1, sem_a2, sem_a3):
    nc = 2
    wid = lax.axis_index("s") * nc + lax.axis_index("c")
    asems = (sem_a0, sem_a1, sem_a2, sem_a3)

    cp_g = pltpu.async_copy(gts_h, gv, sem_g)
    copies = []
    for (base, step, cnt, voff), sem in zip(CHUNKS[:3], asems[:3]):
        start = pl.multiple_of(base + step * wid, 16)
        copies.append(pltpu.async_copy(
            anchor_h.at[pl.ds(start, cnt)], av.at[pl.ds(voff, cnt)], sem))
    small = wid < 20

    @pl.when(small)
    def _():
        start = pl.multiple_of(21504 + 16 * wid, 16)
        pltpu.async_copy(anchor_h.at[pl.ds(start, 16)],
                         av.at[pl.ds(672, 16)], sem_a3).wait()

    cp_g.wait()

    sent_vec = jnp.full((L,), SENT, jnp.float32)
    for k in range(5 * B // L):
        bcx[pl.ds(k * L, L)] = sent_vec

    iota = lax.iota(jnp.int32, L)
    iota4 = iota * 4

    cnts = [jnp.zeros((L,), jnp.int32) for _ in range(5)]
    for k in range(GP // L):
        x0 = plsc.load_gather(gv, [iota4 + (64 * k + 0)])
        y0 = plsc.load_gather(gv, [iota4 + (64 * k + 1)])
        x1 = plsc.load_gather(gv, [iota4 + (64 * k + 2)])
        y1 = plsc.load_gather(gv, [iota4 + (64 * k + 3)])
        area = (x1 - x0) * (y1 - y0)
        lv = jnp.zeros((L,), jnp.float32)
        for thr in (1024.0, 4096.0, 16384.0, 65536.0):
            lv = lv + jnp.where(area >= thr, 1.0, 0.0).astype(jnp.float32)
        lv = jnp.where(area >= 262144.0, 0.0, lv)
        cx = (x0 + x1) * 0.5
        cy = (y0 + y1) * 0.5
        gidx = iota + 16 * k
        valid = gidx < NG
        for l in range(5):
            m = (lv == float(l)) & valid
            r = plsc.cumsum(m.astype(jnp.int32))
            dest = cnts[l] + r + (B * l - 1)
            plsc.store_scatter(bcx, [dest], cx, mask=m)
            plsc.store_scatter(bcy, [dest], cy, mask=m)
            plsc.store_scatter(bgi, [dest], gidx, mask=m)
            cnts[l] = cnts[l] + plsc.all_reduce_population_count(m)
    c = [jnp.max(cnts[l]) for l in range(5)]

    for handle in copies:
        handle.wait()

    neg2 = jnp.full((L,), -2, jnp.int32)

    def scan_group(slots, base, n, unroll):
        boxes = []
        for i in slots:
            row = iota + 16 * i
            boxes.append([plsc.load_gather(av, [row, jnp.full((L,), cc, jnp.int32)])
                          for cc in range(4)])
        nq = (n + (unroll - 1)) // unroll

        def body(q, assigns):
            out = list(assigns)
            j = base + q * unroll
            for u in range(unroll):
                idx = jnp.full((L,), j + u, jnp.int32)
                bx = plsc.load_gather(bcx, [idx])
                by = plsc.load_gather(bcy, [idx])
                bg = plsc.load_gather(bgi, [idx])
                for si, (a0, a1, a2, a3) in enumerate(boxes):
                    m = (bx > a0) & (by > a1) & (bx < a2) & (by < a3)
                    out[si] = jnp.where(m, bg, out[si])
            return tuple(out)

        assigns = lax.fori_loop(0, nq, body, tuple(neg2 for _ in slots))
        for si, i in enumerate(slots):
            outv[pl.ds(16 * i, L)] = assigns[si]

    for g0 in range(0, 32, 4):
        scan_group(list(range(g0, g0 + 4)), 0 * B, c[0], 2)
    scan_group([32, 33, 34, 35], 1 * B, c[1], 2)
    scan_group([36, 37, 38, 39], 1 * B, c[1], 2)
    scan_group([40, 41], 2 * B, c[2], 2)

    @pl.when(small)
    def _():
        is3 = wid < 16
        base = jnp.where(is3, 3 * B, 4 * B)
        n = jnp.where(is3, c[3], c[4])
        scan_group([42], base, n, 4)

    pltpu.sync_copy(outv.at[pl.ds(0, 512)],
                    out_h.at[pl.ds(pl.multiple_of(512 * wid, 16), 512)])
    pltpu.sync_copy(outv.at[pl.ds(512, 128)],
                    out_h.at[pl.ds(pl.multiple_of(16384 + 128 * wid, 16), 128)])
    pltpu.sync_copy(outv.at[pl.ds(640, 32)],
                    out_h.at[pl.ds(pl.multiple_of(20480 + 32 * wid, 16), 32)])

    @pl.when(small)
    def _():
        pltpu.sync_copy(outv.at[pl.ds(672, 16)],
                        out_h.at[pl.ds(pl.multiple_of(21504 + 16 * wid, 16), 16)])


@jax.jit
def kernel(anchor, gts):
    gflat = jnp.full((GF,), SENT, jnp.float32).at[: 4 * gts.shape[0]].set(
        gts.reshape(-1))

    mesh = plsc.VectorSubcoreMesh(core_axis_name="c", subcore_axis_name="s")
    run = pl.kernel(
        _sc_body,
        mesh=mesh,
        compiler_params=pltpu.CompilerParams(needs_layout_passes=False),
        out_type=jax.ShapeDtypeStruct((N,), jnp.int32),
        scratch_types=[
            pltpu.VMEM((PER_TILE, 4), jnp.float32),
            pltpu.VMEM((GF,), jnp.float32),
            pltpu.VMEM((5 * B,), jnp.float32),
            pltpu.VMEM((5 * B,), jnp.float32),
            pltpu.VMEM((5 * B,), jnp.int32),
            pltpu.VMEM((PER_TILE,), jnp.int32),
            pltpu.SemaphoreType.DMA,
            pltpu.SemaphoreType.DMA,
            pltpu.SemaphoreType.DMA,
            pltpu.SemaphoreType.DMA,
            pltpu.SemaphoreType.DMA,
        ],
    )
    return run(anchor, gflat).astype(jnp.int64)


# speedup vs baseline: 1.8587x; 1.0019x over previous
import jax
import jax.numpy as jnp
from jax import lax
from jax.experimental import pallas as pl
from jax.experimental.pallas import tpu as pltpu
from jax.experimental.pallas import tpu_sc as plsc

L = 16
NW = 32
N = 21824
NG = 200
GP = 208
GF = 4 * GP
B = 224
PER_TILE = 688
SENT = 2.0e9

CHUNKS = ((0, 512, 512, 0), (16384, 128, 128, 512),
          (20480, 32, 32, 640), (21504, 16, 16, 672))


def _sc_body(anchor_h, gts_h, out_h,
             av, gv, bcx, bcy, bgi, outv,
             sem_g, sem_a0, sem_a1, sem_a2, sem_a3):
    nc = 2
    wid = lax.axis_index("s") * nc + lax.axis_index("c")
    asems = (sem_a0, sem_a1, sem_a2, sem_a3)

    cp_g = pltpu.async_copy(gts_h, gv, sem_g)
    copies = []
    for (base, step, cnt, voff), sem in zip(CHUNKS[:3], asems[:3]):
        start = pl.multiple_of(base + step * wid, 16)
        copies.append(pltpu.async_copy(
            anchor_h.at[pl.ds(start, cnt)], av.at[pl.ds(voff, cnt)], sem))
    small = wid < 20

    @pl.when(small)
    def _():
        start = pl.multiple_of(21504 + 16 * wid, 16)
        pltpu.async_copy(anchor_h.at[pl.ds(start, 16)],
                         av.at[pl.ds(672, 16)], sem_a3).wait()

    cp_g.wait()

    sent_vec = jnp.full((L,), SENT, jnp.float32)
    for k in range(5 * B // L):
        bcx[pl.ds(k * L, L)] = sent_vec

    iota = lax.iota(jnp.int32, L)
    iota4 = iota * 4

    cnts = [jnp.zeros((L,), jnp.int32) for _ in range(5)]
    for k in range(GP // L):
        x0 = plsc.load_gather(gv, [iota4 + (64 * k + 0)])
        y0 = plsc.load_gather(gv, [iota4 + (64 * k + 1)])
        x1 = plsc.load_gather(gv, [iota4 + (64 * k + 2)])
        y1 = plsc.load_gather(gv, [iota4 + (64 * k + 3)])
        area = (x1 - x0) * (y1 - y0)
        lv = jnp.zeros((L,), jnp.float32)
        for thr in (1024.0, 4096.0, 16384.0, 65536.0):
            lv = lv + jnp.where(area >= thr, 1.0, 0.0).astype(jnp.float32)
        lv = jnp.where(area >= 262144.0, 0.0, lv)
        cx = (x0 + x1) * 0.5
        cy = (y0 + y1) * 0.5
        gidx = iota + 16 * k
        valid = gidx < NG
        for l in range(5):
            m = (lv == float(l)) & valid
            r = plsc.cumsum(m.astype(jnp.int32))
            dest = cnts[l] + r + (B * l - 1)
            plsc.store_scatter(bcx, [dest], cx, mask=m)
            plsc.store_scatter(bcy, [dest], cy, mask=m)
            plsc.store_scatter(bgi, [dest], gidx, mask=m)
            cnts[l] = cnts[l] + plsc.all_reduce_population_count(m)
    c = [jnp.max(cnts[l]) for l in range(5)]

    for handle in copies:
        handle.wait()

    neg2 = jnp.full((L,), -2, jnp.int32)

    def scan_group(slots, base, n, unroll):
        boxes = []
        for i in slots:
            row = iota + 16 * i
            boxes.append([plsc.load_gather(av, [row, jnp.full((L,), cc, jnp.int32)])
                          for cc in range(4)])
        nq = (n + (unroll - 1)) // unroll

        def body(q, assigns):
            out = list(assigns)
            j = base + q * unroll
            for u in range(unroll):
                idx = jnp.full((L,), j + u, jnp.int32)
                bx = plsc.load_gather(bcx, [idx])
                by = plsc.load_gather(bcy, [idx])
                bg = plsc.load_gather(bgi, [idx])
                for si, (a0, a1, a2, a3) in enumerate(boxes):
                    m = (bx > a0) & (by > a1) & (bx < a2) & (by < a3)
                    out[si] = jnp.where(m, bg, out[si])
            return tuple(out)

        assigns = lax.fori_loop(0, nq, body, tuple(neg2 for _ in slots))
        for si, i in enumerate(slots):
            outv[pl.ds(16 * i, L)] = assigns[si]

    for g0 in range(0, 32, 4):
        scan_group(list(range(g0, g0 + 4)), 0 * B, c[0], 2)
    scan_group([32, 33, 34, 35], 1 * B, c[1], 2)
    scan_group([36, 37, 38, 39], 1 * B, c[1], 2)
    scan_group([40, 41], 2 * B, c[2], 2)

    @pl.when(small)
    def _():
        is3 = wid < 16
        base = jnp.where(is3, 3 * B, 4 * B)
        n = jnp.where(is3, c[3], c[4])
        scan_group([42], base, n, 4)

    pltpu.sync_copy(outv.at[pl.ds(0, 512)],
                    out_h.at[pl.ds(pl.multiple_of(512 * wid, 16), 512)])
    pltpu.sync_copy(outv.at[pl.ds(512, 128)],
                    out_h.at[pl.ds(pl.multiple_of(16384 + 128 * wid, 16), 128)])
    pltpu.sync_copy(outv.at[pl.ds(640, 32)],
                    out_h.at[pl.ds(pl.multiple_of(20480 + 32 * wid, 16), 32)])

    @pl.when(small)
    def _():
        pltpu.sync_copy(outv.at[pl.ds(672, 16)],
                        out_h.at[pl.ds(pl.multiple_of(21504 + 16 * wid, 16), 16)])


@jax.jit
def kernel(anchor, gts):
    gflat = jnp.full((GF,), SENT, jnp.float32).at[: 4 * gts.shape[0]].set(
        gts.reshape(-1))

    mesh = plsc.VectorSubcoreMesh(core_axis_name="c", subcore_axis_name="s")
    run = pl.kernel(
        _sc_body,
        mesh=mesh,
        compiler_params=pltpu.CompilerParams(needs_layout_passes=False,
                                             skip_device_barrier=True),
        out_type=jax.ShapeDtypeStruct((N,), jnp.int32),
        scratch_types=[
            pltpu.VMEM((PER_TILE, 4), jnp.float32),
            pltpu.VMEM((GF,), jnp.float32),
            pltpu.VMEM((5 * B,), jnp.float32),
            pltpu.VMEM((5 * B,), jnp.float32),
            pltpu.VMEM((5 * B,), jnp.int32),
            pltpu.VMEM((PER_TILE,), jnp.int32),
            pltpu.SemaphoreType.DMA,
            pltpu.SemaphoreType.DMA,
            pltpu.SemaphoreType.DMA,
            pltpu.SemaphoreType.DMA,
            pltpu.SemaphoreType.DMA,
        ],
    )
    return run(anchor, gflat).astype(jnp.int64)
